# Initial kernel scaffold; baseline (speedup 1.0000x reference)
import functools

import jax
import jax.numpy as jnp
from jax import lax
from jax.experimental import pallas as pl
from jax.experimental.pallas import tpu as pltpu
from jax.experimental.pallas import tpu_sc as plsc

NC = 2
NS = 16
NW = NC * NS
LANES = 16

BLK = 128
NBUF = 8


def _make_gather(N, L, D):
    chunk = N // NW
    nsteps = chunk // (BLK * NBUF)
    mesh = plsc.VectorSubcoreMesh(core_axis_name="c", subcore_axis_name="s")

    scratch = (
        [pltpu.VMEM((chunk,), jnp.int32)]
        + [pltpu.VMEM((BLK,), jnp.int32) for _ in range(NBUF)]
        + [pltpu.VMEM((BLK, D), jnp.float32) for _ in range(NBUF)]
        + [pltpu.SemaphoreType.DMA for _ in range(2 * NBUF)]
    )

    @functools.partial(
        pl.kernel,
        out_type=jax.ShapeDtypeStruct((N, D), jnp.float32),
        mesh=mesh,
        scratch_types=scratch,
    )
    def gather_kernel(reps_hbm, idx_hbm, out_hbm, *sc):
        idx_raw = sc[0]
        idxbs = sc[1:1 + NBUF]
        rowss = sc[1 + NBUF:1 + 2 * NBUF]
        gsems = sc[1 + 2 * NBUF:1 + 3 * NBUF]
        osems = sc[1 + 3 * NBUF:1 + 4 * NBUF]

        wid = lax.axis_index("s") * NC + lax.axis_index("c")
        base = wid * chunk
        pltpu.sync_copy(idx_hbm.at[pl.ds(base, chunk)], idx_raw)
        iota = lax.iota(jnp.int32, 16)

        def step(ss, carry):
            sbase = ss * (NBUF * BLK)
            gh = []
            for b in range(NBUF):
                idxb = idxbs[b]
                for j in range(BLK // LANES):
                    off = sbase + b * BLK + j * LANES
                    e = off + iota
                    raw = idx_raw[pl.ds(off, LANES)]
                    idxb[pl.ds(j * LANES, LANES)] = base + (e - e % L) + raw
                gh.append(pltpu.async_copy(reps_hbm.at[idxb], rowss[b], gsems[b]))
            oh = []
            for b in range(NBUF):
                gh[b].wait()
                off = base + sbase + b * BLK
                oh.append(pltpu.async_copy(
                    rowss[b], out_hbm.at[pl.ds(off, BLK)], osems[b]))
            for b in range(NBUF):
                oh[b].wait()
            return carry

        lax.fori_loop(0, nsteps, step, 0)

    return gather_kernel


def kernel(reps, index):
    B, L, D = reps.shape
    N = B * L
    reps_flat = reps.reshape(N, D)
    idx_flat = index.reshape(N).astype(jnp.int32)
    out_flat = _make_gather(N, L, D)(reps_flat, idx_flat)
    return out_flat.reshape(B, L, D)


# speedup vs baseline: 3.1631x; 3.1631x over previous
import functools

import jax
import jax.numpy as jnp
from jax import lax
from jax.experimental import pallas as pl
from jax.experimental.pallas import tpu as pltpu
from jax.experimental.pallas import tpu_sc as plsc

NC = 2
NS = 16
NW = NC * NS
LANES = 16

BLK = 128
NBUF = 8


def _make_gather(N, L, D):
    chunk = N // NW
    nsteps = chunk // (BLK * NBUF)
    mesh = plsc.VectorSubcoreMesh(core_axis_name="c", subcore_axis_name="s")

    scratch = (
        [pltpu.VMEM((chunk,), jnp.int32)]
        + [pltpu.VMEM((BLK,), jnp.int32) for _ in range(NBUF)]
        + [pltpu.VMEM((BLK, D), jnp.float32) for _ in range(NBUF)]
        + [pltpu.SemaphoreType.DMA for _ in range(2 * NBUF)]
    )

    @functools.partial(
        pl.kernel,
        out_type=jax.ShapeDtypeStruct((N, D), jnp.float32),
        mesh=mesh,
        scratch_types=scratch,
        compiler_params=pltpu.CompilerParams(use_tc_tiling_on_sc=False),
    )
    def gather_kernel(reps_hbm, idx_hbm, out_hbm, *sc):
        idx_raw = sc[0]
        idxbs = sc[1:1 + NBUF]
        rowss = sc[1 + NBUF:1 + 2 * NBUF]
        gsems = sc[1 + 2 * NBUF:1 + 3 * NBUF]
        osems = sc[1 + 3 * NBUF:1 + 4 * NBUF]

        wid = lax.axis_index("s") * NC + lax.axis_index("c")
        base = wid * chunk
        pltpu.sync_copy(idx_hbm.at[pl.ds(base, chunk)], idx_raw)
        iota = lax.iota(jnp.int32, 16)

        def step(ss, carry):
            sbase = ss * (NBUF * BLK)
            gh = []
            for b in range(NBUF):
                idxb = idxbs[b]
                for j in range(BLK // LANES):
                    off = sbase + b * BLK + j * LANES
                    e = off + iota
                    raw = idx_raw[pl.ds(off, LANES)]
                    idxb[pl.ds(j * LANES, LANES)] = base + (e - e % L) + raw
                gh.append(pltpu.async_copy(reps_hbm.at[idxb], rowss[b], gsems[b]))
            oh = []
            for b in range(NBUF):
                gh[b].wait()
                off = base + sbase + b * BLK
                oh.append(pltpu.async_copy(
                    rowss[b], out_hbm.at[pl.ds(off, BLK)], osems[b]))
            for b in range(NBUF):
                oh[b].wait()
            return carry

        lax.fori_loop(0, nsteps, step, 0)

    return gather_kernel


def kernel(reps, index):
    B, L, D = reps.shape
    N = B * L
    reps_flat = reps.reshape(N, D)
    idx_flat = index.reshape(N).astype(jnp.int32)
    out_flat = _make_gather(N, L, D)(reps_flat, idx_flat)
    return out_flat.reshape(B, L, D)


# speedup vs baseline: 7.5289x; 2.3802x over previous
import functools

import jax
import jax.numpy as jnp
from jax import lax
from jax.experimental import pallas as pl
from jax.experimental.pallas import tpu as pltpu
from jax.experimental.pallas import tpu_sc as plsc

NC, NS, LANES = 2, 16, 16

LCH = 10
DW = 2


def _make_k(Ldim, DT, BT, D8, B128):
    npairs = BT // NC
    nch = Ldim // LCH
    mesh = plsc.VectorSubcoreMesh(core_axis_name="c", subcore_axis_name="s")
    scratch = [
        pltpu.VMEM((Ldim, DW, B128), jnp.float32),
        pltpu.VMEM((Ldim, DW, B128), jnp.float32),
        pltpu.VMEM((LCH, B128), jnp.int32),
        pltpu.VMEM((LCH, B128), jnp.int32),
        pltpu.VMEM((LCH, DW, B128), jnp.float32),
        pltpu.VMEM((LCH, DW, B128), jnp.float32),
        pltpu.SemaphoreType.DMA,
        pltpu.SemaphoreType.DMA,
        pltpu.SemaphoreType.DMA,
        pltpu.SemaphoreType.DMA,
        pltpu.SemaphoreType.DMA,
        pltpu.SemaphoreType.DMA,
    ]

    @functools.partial(
        pl.kernel,
        out_type=jax.ShapeDtypeStruct((Ldim, DT, BT, D8, B128), jnp.float32),
        mesh=mesh,
        scratch_types=scratch,
        compiler_params=pltpu.CompilerParams(
            use_tc_tiling_on_sc=False, needs_layout_passes=False),
    )
    def k(rt5, it2, out5, stgA, stgB, idxA, idxB, outA, outB,
          stgAs, stgBs, idxAs, idxBs, outAs, outBs):
        cc = lax.axis_index("c")
        ss = lax.axis_index("s")
        dt = ss // 2
        spair = ss % 2
        iota = lax.iota(jnp.int32, LANES)
        bvecs = [j * LANES + iota for j in range(B128 // LANES)]
        djvecs = [jnp.zeros((LANES,), jnp.int32) + dj for dj in range(DW)]

        def stg_src(pair, d80):
            bt = cc * npairs + pair
            return rt5.at[:, dt, bt, pl.ds(d80, DW), :]

        def idx_src(bt, chunk_l0):
            return it2.at[pl.ds(chunk_l0, LCH), pl.ds(bt * B128, B128)]

        def out_dst(bt, d80, l0):
            return out5.at[pl.ds(l0, LCH), dt, bt, pl.ds(d80, DW), :]

        def compute_chunk(stg, idxbuf, outbuf):
            for li in range(LCH):
                for j in range(B128 // LANES):
                    g16 = idxbuf[li, pl.ds(j * LANES, LANES)]
                    for dj in range(DW):
                        v = plsc.load_gather(stg, [g16, djvecs[dj], bvecs[j]])
                        outbuf[li, dj, pl.ds(j * LANES, LANES)] = v

        def unit_compute(pair, bt, d80, stg, next_bt, next_valid, is_first_unit):
            def body(t, carry):
                for kpar, (idxbuf, isem, outbuf, osem) in enumerate(
                        [(idxA, idxAs, outA, outAs), (idxB, idxBs, outB, outBs)]):
                    kchunk = 2 * t + kpar
                    l0 = kchunk * LCH
                    pltpu.make_async_copy(idx_src(bt, l0), idxbuf, isem).wait()
                    if is_first_unit:
                        skip = jnp.logical_and(pair == 0, t == 0)
                    else:
                        skip = jnp.logical_and(pair < 0, t == 0)

                    @pl.when(jnp.logical_not(skip))
                    def _():
                        pltpu.make_async_copy(outbuf, out_dst(bt, d80, l0), osem).wait()

                    compute_chunk(stg, idxbuf, outbuf)
                    pltpu.async_copy(outbuf, out_dst(bt, d80, l0), osem)

                    @pl.when(t < nch // 2 - 1)
                    def _():
                        pltpu.async_copy(idx_src(bt, l0 + 2 * LCH), idxbuf, isem)

                    @pl.when(jnp.logical_and(t == nch // 2 - 1, next_valid))
                    def _():
                        pltpu.async_copy(idx_src(next_bt, kpar * LCH), idxbuf, isem)
                return carry

            lax.fori_loop(0, nch // 2, body, 0)

        d8A = spair * 2 * DW
        d8B = (spair * 2 + 1) * DW
        pltpu.async_copy(stg_src(0, d8A), stgA, stgAs)
        pltpu.async_copy(stg_src(0, d8B), stgB, stgBs)
        bt0 = cc * npairs
        pltpu.async_copy(idx_src(bt0, 0), idxA, idxAs)
        pltpu.async_copy(idx_src(bt0, LCH), idxB, idxBs)

        def pair_body(pair, carry):
            bt = cc * npairs + pair
            has_next = pair + 1 < npairs
            pltpu.make_async_copy(stg_src(pair, d8A), stgA, stgAs).wait()
            unit_compute(pair, bt, d8A, stgA,
                         next_bt=bt, next_valid=jnp.bool_(True),
                         is_first_unit=True)

            @pl.when(has_next)
            def _():
                pltpu.async_copy(stg_src(pair + 1, d8A), stgA, stgAs)

            pltpu.make_async_copy(stg_src(pair, d8B), stgB, stgBs).wait()
            unit_compute(pair, bt, d8B, stgB,
                         next_bt=bt + 1, next_valid=has_next,
                         is_first_unit=False)

            @pl.when(has_next)
            def _():
                pltpu.async_copy(stg_src(pair + 1, d8B), stgB, stgBs)

            return carry

        lax.fori_loop(0, npairs, pair_body, 0)

        lastb = cc * npairs + npairs - 1
        l_last = (nch - 2) * LCH
        pltpu.make_async_copy(outA, out_dst(lastb, d8B, l_last), outAs).wait()
        pltpu.make_async_copy(outB, out_dst(lastb, d8B, l_last + LCH), outBs).wait()

    return k


def kernel(reps, index):
    B, L, D = reps.shape
    DT, D8, B128 = D // 8, 8, 128
    BT = B // B128
    rt = jnp.transpose(reps, (1, 2, 0))
    rt5 = rt.reshape(L, DT, D8, BT, B128)
    rt5 = jnp.transpose(rt5, (0, 1, 3, 2, 4))
    it2 = jnp.transpose(index.reshape(B, L).astype(jnp.int32), (1, 0))
    out5 = _make_k(L, DT, BT, D8, B128)(rt5, it2)
    out = jnp.transpose(out5, (0, 1, 3, 2, 4)).reshape(L, D, B)
    return jnp.transpose(out, (2, 0, 1))


# speedup vs baseline: 9.6175x; 1.2774x over previous
import functools

import jax
import jax.numpy as jnp
from jax import lax
from jax.experimental import pallas as pl
from jax.experimental.pallas import tpu as pltpu
from jax.experimental.pallas import tpu_sc as plsc

NC, NS, LANES = 2, 16, 16

LCH = 10
DW = 2


def _make_k(Ldim, DT, BT, D8, B128):
    npairs = BT // NC
    nch = Ldim // LCH
    mesh = plsc.VectorSubcoreMesh(core_axis_name="c", subcore_axis_name="s")
    scratch = [
        pltpu.VMEM((Ldim, DW, B128), jnp.float32),
        pltpu.VMEM((Ldim, DW, B128), jnp.float32),
        pltpu.VMEM((LCH, B128), jnp.int32),
        pltpu.VMEM((LCH, B128), jnp.int32),
        pltpu.VMEM((LCH, DW, B128), jnp.float32),
        pltpu.VMEM((LCH, DW, B128), jnp.float32),
        pltpu.SemaphoreType.DMA,
        pltpu.SemaphoreType.DMA,
        pltpu.SemaphoreType.DMA,
        pltpu.SemaphoreType.DMA,
        pltpu.SemaphoreType.DMA,
        pltpu.SemaphoreType.DMA,
    ]

    @functools.partial(
        pl.kernel,
        out_type=jax.ShapeDtypeStruct((Ldim, DT, BT, D8, B128), jnp.float32),
        mesh=mesh,
        scratch_types=scratch,
        compiler_params=pltpu.CompilerParams(
            use_tc_tiling_on_sc=False, needs_layout_passes=False),
    )
    def k(rt5, it2, out5, stgA, stgB, idxA, idxB, outA, outB,
          stgAs, stgBs, idxAs, idxBs, outAs, outBs):
        cc = lax.axis_index("c")
        ss = lax.axis_index("s")
        dt = ss // 2
        spair = ss % 2
        iota = lax.iota(jnp.int32, LANES)
        bvecs = [j * LANES + iota for j in range(B128 // LANES)]
        djvecs = [jnp.zeros((LANES,), jnp.int32) + dj for dj in range(DW)]

        def stg_src(pair, d80):
            bt = cc * npairs + pair
            return rt5.at[:, dt, bt, pl.ds(d80, DW), :]

        def idx_src(bt, chunk_l0):
            return it2.at[pl.ds(chunk_l0, LCH), pl.ds(bt * B128, B128)]

        def out_dst(bt, d80, l0):
            return out5.at[pl.ds(l0, LCH), dt, bt, pl.ds(d80, DW), :]

        def compute_chunk(stg, idxbuf, outbuf):
            nj = B128 // LANES
            for li in range(LCH):
                gs = [idxbuf[li, pl.ds(j * LANES, LANES)] for j in range(nj)]
                vals = [(j, dj, plsc.load_gather(stg, [gs[j], djvecs[dj], bvecs[j]]))
                        for j in range(nj) for dj in range(DW)]
                for j, dj, v in vals:
                    outbuf[li, dj, pl.ds(j * LANES, LANES)] = v

        def unit_compute(pair, bt, d80, stg, next_bt, next_valid, is_first_unit):
            def body(t, carry):
                for kpar, (idxbuf, isem, outbuf, osem) in enumerate(
                        [(idxA, idxAs, outA, outAs), (idxB, idxBs, outB, outBs)]):
                    kchunk = 2 * t + kpar
                    l0 = kchunk * LCH
                    pltpu.make_async_copy(idx_src(bt, l0), idxbuf, isem).wait()
                    if is_first_unit:
                        skip = jnp.logical_and(pair == 0, t == 0)
                    else:
                        skip = jnp.logical_and(pair < 0, t == 0)

                    @pl.when(jnp.logical_not(skip))
                    def _():
                        pltpu.make_async_copy(outbuf, out_dst(bt, d80, l0), osem).wait()

                    compute_chunk(stg, idxbuf, outbuf)
                    pltpu.async_copy(outbuf, out_dst(bt, d80, l0), osem)

                    @pl.when(t < nch // 2 - 1)
                    def _():
                        pltpu.async_copy(idx_src(bt, l0 + 2 * LCH), idxbuf, isem)

                    @pl.when(jnp.logical_and(t == nch // 2 - 1, next_valid))
                    def _():
                        pltpu.async_copy(idx_src(next_bt, kpar * LCH), idxbuf, isem)
                return carry

            lax.fori_loop(0, nch // 2, body, 0)

        d8A = spair * 2 * DW
        d8B = (spair * 2 + 1) * DW
        pltpu.async_copy(stg_src(0, d8A), stgA, stgAs)
        pltpu.async_copy(stg_src(0, d8B), stgB, stgBs)
        bt0 = cc * npairs
        pltpu.async_copy(idx_src(bt0, 0), idxA, idxAs)
        pltpu.async_copy(idx_src(bt0, LCH), idxB, idxBs)

        def pair_body(pair, carry):
            bt = cc * npairs + pair
            has_next = pair + 1 < npairs
            pltpu.make_async_copy(stg_src(pair, d8A), stgA, stgAs).wait()
            unit_compute(pair, bt, d8A, stgA,
                         next_bt=bt, next_valid=jnp.bool_(True),
                         is_first_unit=True)

            @pl.when(has_next)
            def _():
                pltpu.async_copy(stg_src(pair + 1, d8A), stgA, stgAs)

            pltpu.make_async_copy(stg_src(pair, d8B), stgB, stgBs).wait()
            unit_compute(pair, bt, d8B, stgB,
                         next_bt=bt + 1, next_valid=has_next,
                         is_first_unit=False)

            @pl.when(has_next)
            def _():
                pltpu.async_copy(stg_src(pair + 1, d8B), stgB, stgBs)

            return carry

        lax.fori_loop(0, npairs, pair_body, 0)

        lastb = cc * npairs + npairs - 1
        l_last = (nch - 2) * LCH
        pltpu.make_async_copy(outA, out_dst(lastb, d8B, l_last), outAs).wait()
        pltpu.make_async_copy(outB, out_dst(lastb, d8B, l_last + LCH), outBs).wait()

    return k


def kernel(reps, index):
    B, L, D = reps.shape
    DT, D8, B128 = D // 8, 8, 128
    BT = B // B128
    rt = jnp.transpose(reps, (1, 2, 0))
    rt5 = rt.reshape(L, DT, D8, BT, B128)
    rt5 = jnp.transpose(rt5, (0, 1, 3, 2, 4))
    it2 = jnp.transpose(index.reshape(B, L).astype(jnp.int32), (1, 0))
    out5 = _make_k(L, DT, BT, D8, B128)(rt5, it2)
    out = jnp.transpose(out5, (0, 1, 3, 2, 4)).reshape(L, D, B)
    return jnp.transpose(out, (2, 0, 1))


# speedup vs baseline: 12.4565x; 1.2952x over previous
import functools

import jax
import jax.numpy as jnp
from jax import lax
from jax.experimental import pallas as pl
from jax.experimental.pallas import tpu as pltpu
from jax.experimental.pallas import tpu_sc as plsc

NC, NS, LANES = 2, 16, 16

LCH = 20
DW = 2


def _make_k(Ldim, DT, BT, D8, B128):
    npairs = BT // NC
    nch = Ldim // LCH
    mesh = plsc.VectorSubcoreMesh(core_axis_name="c", subcore_axis_name="s")
    scratch = [
        pltpu.VMEM((Ldim, DW, B128), jnp.float32),
        pltpu.VMEM((Ldim, DW, B128), jnp.float32),
        pltpu.VMEM((LCH, B128), jnp.int32),
        pltpu.VMEM((LCH, B128), jnp.int32),
        pltpu.VMEM((LCH, DW, B128), jnp.float32),
        pltpu.VMEM((LCH, DW, B128), jnp.float32),
        pltpu.SemaphoreType.DMA,
        pltpu.SemaphoreType.DMA,
        pltpu.SemaphoreType.DMA,
        pltpu.SemaphoreType.DMA,
        pltpu.SemaphoreType.DMA,
        pltpu.SemaphoreType.DMA,
    ]

    @functools.partial(
        pl.kernel,
        out_type=jax.ShapeDtypeStruct((Ldim, DT, BT, D8, B128), jnp.float32),
        mesh=mesh,
        scratch_types=scratch,
        compiler_params=pltpu.CompilerParams(
            use_tc_tiling_on_sc=False, needs_layout_passes=False),
    )
    def k(rt5, it2, out5, stgA, stgB, idxA, idxB, outA, outB,
          stgAs, stgBs, idxAs, idxBs, outAs, outBs):
        cc = lax.axis_index("c")
        ss = lax.axis_index("s")
        dt = ss // 2
        spair = ss % 2
        iota = lax.iota(jnp.int32, LANES)
        bvecs = [j * LANES + iota for j in range(B128 // LANES)]
        djvecs = [jnp.zeros((LANES,), jnp.int32) + dj for dj in range(DW)]

        def stg_src(pair, d80):
            bt = cc * npairs + pair
            return rt5.at[:, dt, bt, pl.ds(d80, DW), :]

        def idx_src(bt, chunk_l0):
            return it2.at[pl.ds(chunk_l0, LCH), pl.ds(bt * B128, B128)]

        def out_dst(bt, d80, l0):
            return out5.at[pl.ds(l0, LCH), dt, bt, pl.ds(d80, DW), :]

        def compute_chunk(stg, idxbuf, outbuf):
            nj = B128 // LANES
            stages = [(li, jh) for li in range(LCH) for jh in range(2)]
            prev = None
            for li, jh in stages:
                jset = range(jh * nj // 2, (jh + 1) * nj // 2)
                gs = {j: idxbuf[li, pl.ds(j * LANES, LANES)] for j in jset}
                vals = [(li, j, dj,
                         plsc.load_gather(stg, [gs[j], djvecs[dj], bvecs[j]]))
                        for j in jset for dj in range(DW)]
                if prev is not None:
                    for pli, pj, pdj, pv in prev:
                        outbuf[pli, pdj, pl.ds(pj * LANES, LANES)] = pv
                prev = vals
            for pli, pj, pdj, pv in prev:
                outbuf[pli, pdj, pl.ds(pj * LANES, LANES)] = pv

        def unit_compute(pair, bt, d80, stg, next_bt, next_valid, is_first_unit):
            def body(t, carry):
                for kpar, (idxbuf, isem, outbuf, osem) in enumerate(
                        [(idxA, idxAs, outA, outAs), (idxB, idxBs, outB, outBs)]):
                    kchunk = 2 * t + kpar
                    l0 = kchunk * LCH
                    pltpu.make_async_copy(idx_src(bt, l0), idxbuf, isem).wait()
                    if is_first_unit:
                        skip = jnp.logical_and(pair == 0, t == 0)
                    else:
                        skip = jnp.logical_and(pair < 0, t == 0)

                    @pl.when(jnp.logical_not(skip))
                    def _():
                        pltpu.make_async_copy(outbuf, out_dst(bt, d80, l0), osem).wait()

                    compute_chunk(stg, idxbuf, outbuf)
                    pltpu.async_copy(outbuf, out_dst(bt, d80, l0), osem)

                    @pl.when(t < nch // 2 - 1)
                    def _():
                        pltpu.async_copy(idx_src(bt, l0 + 2 * LCH), idxbuf, isem)

                    @pl.when(jnp.logical_and(t == nch // 2 - 1, next_valid))
                    def _():
                        pltpu.async_copy(idx_src(next_bt, kpar * LCH), idxbuf, isem)
                return carry

            lax.fori_loop(0, nch // 2, body, 0)

        d8A = spair * 2 * DW
        d8B = (spair * 2 + 1) * DW
        pltpu.async_copy(stg_src(0, d8A), stgA, stgAs)
        pltpu.async_copy(stg_src(0, d8B), stgB, stgBs)
        bt0 = cc * npairs
        pltpu.async_copy(idx_src(bt0, 0), idxA, idxAs)
        pltpu.async_copy(idx_src(bt0, LCH), idxB, idxBs)

        def pair_body(pair, carry):
            bt = cc * npairs + pair
            has_next = pair + 1 < npairs
            pltpu.make_async_copy(stg_src(pair, d8A), stgA, stgAs).wait()
            unit_compute(pair, bt, d8A, stgA,
                         next_bt=bt, next_valid=jnp.bool_(True),
                         is_first_unit=True)

            @pl.when(has_next)
            def _():
                pltpu.async_copy(stg_src(pair + 1, d8A), stgA, stgAs)

            pltpu.make_async_copy(stg_src(pair, d8B), stgB, stgBs).wait()
            unit_compute(pair, bt, d8B, stgB,
                         next_bt=bt + 1, next_valid=has_next,
                         is_first_unit=False)

            @pl.when(has_next)
            def _():
                pltpu.async_copy(stg_src(pair + 1, d8B), stgB, stgBs)

            return carry

        lax.fori_loop(0, npairs, pair_body, 0)

        lastb = cc * npairs + npairs - 1
        l_last = (nch - 2) * LCH
        pltpu.make_async_copy(outA, out_dst(lastb, d8B, l_last), outAs).wait()
        pltpu.make_async_copy(outB, out_dst(lastb, d8B, l_last + LCH), outBs).wait()

    return k


def kernel(reps, index):
    B, L, D = reps.shape
    DT, D8, B128 = D // 8, 8, 128
    BT = B // B128
    rt = jnp.transpose(reps, (1, 2, 0))
    rt5 = rt.reshape(L, DT, D8, BT, B128)
    rt5 = jnp.transpose(rt5, (0, 1, 3, 2, 4))
    it2 = jnp.transpose(index.reshape(B, L).astype(jnp.int32), (1, 0))
    out5 = _make_k(L, DT, BT, D8, B128)(rt5, it2)
    out = jnp.transpose(out5, (0, 1, 3, 2, 4)).reshape(L, D, B)
    return jnp.transpose(out, (2, 0, 1))


# speedup vs baseline: 12.7597x; 1.0243x over previous
import functools

import jax
import jax.numpy as jnp
from jax import lax
from jax.experimental import pallas as pl
from jax.experimental.pallas import tpu as pltpu
from jax.experimental.pallas import tpu_sc as plsc

NC, NS, LANES = 2, 16, 16

LCH = 20
DW = 2


def _make_k(Ldim, DT, BT, D8, B128):
    npairs = BT // NC
    nch = Ldim // LCH
    mesh = plsc.VectorSubcoreMesh(core_axis_name="c", subcore_axis_name="s")
    scratch = [
        pltpu.VMEM((Ldim, DW, B128), jnp.float32),
        pltpu.VMEM((Ldim, DW, B128), jnp.float32),
        pltpu.VMEM((LCH, B128), jnp.int16),
        pltpu.VMEM((LCH, B128), jnp.int16),
        pltpu.VMEM((LCH, DW, B128), jnp.float32),
        pltpu.VMEM((LCH, DW, B128), jnp.float32),
        pltpu.SemaphoreType.DMA,
        pltpu.SemaphoreType.DMA,
        pltpu.SemaphoreType.DMA,
        pltpu.SemaphoreType.DMA,
        pltpu.SemaphoreType.DMA,
        pltpu.SemaphoreType.DMA,
    ]

    @functools.partial(
        pl.kernel,
        out_type=jax.ShapeDtypeStruct((Ldim, DT, BT, D8, B128), jnp.float32),
        mesh=mesh,
        scratch_types=scratch,
        compiler_params=pltpu.CompilerParams(
            use_tc_tiling_on_sc=False, needs_layout_passes=False),
    )
    def k(rt5, it2, out5, stgA, stgB, idxA, idxB, outA, outB,
          stgAs, stgBs, idxAs, idxBs, outAs, outBs):
        cc = lax.axis_index("c")
        ss = lax.axis_index("s")
        dt = ss // 2
        spair = ss % 2
        iota = lax.iota(jnp.int32, LANES)
        bvecs = [j * LANES + iota for j in range(B128 // LANES)]
        djvecs = [jnp.zeros((LANES,), jnp.int32) + dj for dj in range(DW)]

        def stg_src(pair, d80):
            bt = cc * npairs + pair
            return rt5.at[:, dt, bt, pl.ds(d80, DW), :]

        def idx_src(bt, chunk_l0):
            return it2.at[pl.ds(chunk_l0, LCH), pl.ds(bt * B128, B128)]

        def out_dst(bt, d80, l0):
            return out5.at[pl.ds(l0, LCH), dt, bt, pl.ds(d80, DW), :]

        def compute_chunk(stg, idxbuf, outbuf):
            nj = B128 // LANES
            stages = [(li, jh) for li in range(LCH) for jh in range(2)]
            prev = None
            for li, jh in stages:
                mset = range(jh * nj // 4, (jh + 1) * nj // 4)
                gs = {}
                for m in mset:
                    packed = plsc.bitcast(
                        idxbuf[li, pl.ds(m * 2 * LANES, 2 * LANES)], jnp.int32)
                    gs[2 * m] = packed & 0xFFFF
                    gs[2 * m + 1] = lax.shift_right_logical(packed, 16)
                vals = [(li, j, dj,
                         plsc.load_gather(stg, [gs[j], djvecs[dj], bvecs[j]]))
                        for m in mset for j in (2 * m, 2 * m + 1)
                        for dj in range(DW)]
                if prev is not None:
                    for pli, pj, pdj, pv in prev:
                        outbuf[pli, pdj, pl.ds(pj * LANES, LANES)] = pv
                prev = vals
            for pli, pj, pdj, pv in prev:
                outbuf[pli, pdj, pl.ds(pj * LANES, LANES)] = pv

        def unit_compute(pair, bt, d80, stg, next_bt, next_valid, is_first_unit):
            def body(t, carry):
                for kpar, (idxbuf, isem, outbuf, osem) in enumerate(
                        [(idxA, idxAs, outA, outAs), (idxB, idxBs, outB, outBs)]):
                    kchunk = 2 * t + kpar
                    l0 = kchunk * LCH
                    pltpu.make_async_copy(idx_src(bt, l0), idxbuf, isem).wait()
                    if is_first_unit:
                        skip = jnp.logical_and(pair == 0, t == 0)
                    else:
                        skip = jnp.logical_and(pair < 0, t == 0)

                    @pl.when(jnp.logical_not(skip))
                    def _():
                        pltpu.make_async_copy(outbuf, out_dst(bt, d80, l0), osem).wait()

                    compute_chunk(stg, idxbuf, outbuf)
                    pltpu.async_copy(outbuf, out_dst(bt, d80, l0), osem)

                    @pl.when(t < nch // 2 - 1)
                    def _():
                        pltpu.async_copy(idx_src(bt, l0 + 2 * LCH), idxbuf, isem)

                    @pl.when(jnp.logical_and(t == nch // 2 - 1, next_valid))
                    def _():
                        pltpu.async_copy(idx_src(next_bt, kpar * LCH), idxbuf, isem)
                return carry

            lax.fori_loop(0, nch // 2, body, 0)

        d8A = spair * 2 * DW
        d8B = (spair * 2 + 1) * DW
        pltpu.async_copy(stg_src(0, d8A), stgA, stgAs)
        pltpu.async_copy(stg_src(0, d8B), stgB, stgBs)
        bt0 = cc * npairs
        pltpu.async_copy(idx_src(bt0, 0), idxA, idxAs)
        pltpu.async_copy(idx_src(bt0, LCH), idxB, idxBs)

        def pair_body(pair, carry):
            bt = cc * npairs + pair
            has_next = pair + 1 < npairs
            pltpu.make_async_copy(stg_src(pair, d8A), stgA, stgAs).wait()
            unit_compute(pair, bt, d8A, stgA,
                         next_bt=bt, next_valid=jnp.bool_(True),
                         is_first_unit=True)

            @pl.when(has_next)
            def _():
                pltpu.async_copy(stg_src(pair + 1, d8A), stgA, stgAs)

            pltpu.make_async_copy(stg_src(pair, d8B), stgB, stgBs).wait()
            unit_compute(pair, bt, d8B, stgB,
                         next_bt=bt + 1, next_valid=has_next,
                         is_first_unit=False)

            @pl.when(has_next)
            def _():
                pltpu.async_copy(stg_src(pair + 1, d8B), stgB, stgBs)

            return carry

        lax.fori_loop(0, npairs, pair_body, 0)

        lastb = cc * npairs + npairs - 1
        l_last = (nch - 2) * LCH
        pltpu.make_async_copy(outA, out_dst(lastb, d8B, l_last), outAs).wait()
        pltpu.make_async_copy(outB, out_dst(lastb, d8B, l_last + LCH), outBs).wait()

    return k


def kernel(reps, index):
    B, L, D = reps.shape
    DT, D8, B128 = D // 8, 8, 128
    BT = B // B128
    rt = jnp.transpose(reps, (1, 2, 0))
    rt5 = rt.reshape(L, DT, D8, BT, B128)
    rt5 = jnp.transpose(rt5, (0, 1, 3, 2, 4))
    it2 = jnp.transpose(index.reshape(B, L).astype(jnp.int16), (1, 0))
    it2p = (it2.reshape(L, B // 32, 2, LANES)
            .transpose(0, 1, 3, 2)
            .reshape(L, B))
    out5 = _make_k(L, DT, BT, D8, B128)(rt5, it2p)
    out = jnp.transpose(out5, (0, 1, 3, 2, 4)).reshape(L, D, B)
    return jnp.transpose(out, (2, 0, 1))
